# Initial kernel scaffold; baseline (speedup 1.0000x reference)
import jax
import jax.numpy as jnp
from jax.experimental import pallas as pl


def kernel(loc_data, conf_data, priors, targets):
    raise NotImplementedError("write your pallas kernel here")


# speedup vs baseline: 7.5991x; 7.5991x over previous
import functools

import jax
import jax.numpy as jnp
from jax import lax
from jax.experimental import pallas as pl
from jax.experimental.pallas import tpu as pltpu

_NUM_CLASSES = 21
_THRESHOLD = 0.5
_NEGPOS_RATIO = 3
_V0 = 0.1
_V1 = 0.2
_B = 32
_P = 8732
_O = 12

_INTERPRET = False


def _body(loc_ref, conf_ref, priors_ref, targets_ref,
          ll_ref, lc_ref, np_ref):
    b = pl.program_id(0)

    px = priors_ref[0, :]
    py = priors_ref[1, :]
    pw = priors_ref[2, :]
    ph = priors_ref[3, :]
    x1p = px - pw * 0.5
    y1p = py - ph * 0.5
    x2p = px + pw * 0.5
    y2p = py + ph * 0.5
    area_p = pw * ph

    iota = lax.broadcasted_iota(jnp.int32, (_P,), 0)

    NEG1 = jnp.float32(-1.0)
    bv = jnp.full((_P,), NEG1, jnp.float32)
    bti = jnp.zeros((_P,), jnp.int32)
    bpi = []
    ious = []
    for o in range(_O):
        tx1 = targets_ref[0, 0, o]
        ty1 = targets_ref[0, 1, o]
        tx2 = targets_ref[0, 2, o]
        ty2 = targets_ref[0, 3, o]
        area_t = (tx2 - tx1) * (ty2 - ty1)
        iw = jnp.maximum(jnp.minimum(x2p, tx2) - jnp.maximum(x1p, tx1), 0.0)
        ih = jnp.maximum(jnp.minimum(y2p, ty2) - jnp.maximum(y1p, ty1), 0.0)
        inter = iw * ih
        iou = inter / (area_p + area_t - inter)
        ious.append(iou)
        upd = iou > bv
        bti = jnp.where(upd, o, bti)
        bv = jnp.where(upd, iou, bv)
        m = jnp.max(iou)
        bpi.append(jnp.min(jnp.where(iou == m, iota, _P)))

    for o in range(_O):
        mask = iota == bpi[o]
        bv = jnp.where(mask, 2.0, bv)
        bti = jnp.where(mask, o, bti)

    mx1 = jnp.zeros((_P,), jnp.float32)
    my1 = jnp.zeros((_P,), jnp.float32)
    mx2 = jnp.zeros((_P,), jnp.float32)
    my2 = jnp.zeros((_P,), jnp.float32)
    labf = jnp.zeros((_P,), jnp.float32)
    for o in range(_O):
        sel = bti == o
        mx1 = jnp.where(sel, targets_ref[0, 0, o], mx1)
        my1 = jnp.where(sel, targets_ref[0, 1, o], my1)
        mx2 = jnp.where(sel, targets_ref[0, 2, o], mx2)
        my2 = jnp.where(sel, targets_ref[0, 3, o], my2)
        labf = jnp.where(sel, targets_ref[0, 4, o], labf)

    conf_t = jnp.where(bv < _THRESHOLD, 0.0, labf + 1.0)
    pos = conf_t > 0.0
    posf = pos.astype(jnp.float32)
    num_pos = jnp.sum(posf)

    gcx = ((mx1 + mx2) * 0.5 - px) / (_V0 * pw)
    gcy = ((my1 + my2) * 0.5 - py) / (_V0 * ph)
    gw = jnp.log((mx2 - mx1) / pw) * (1.0 / _V1)
    gh = jnp.log((my2 - my1) / ph) * (1.0 / _V1)
    ll = jnp.float32(0.0)
    for c, g in enumerate((gcx, gcy, gw, gh)):
        d = loc_ref[0, c, :] - g
        ad = jnp.abs(d)
        sl1 = jnp.where(ad < 1.0, 0.5 * d * d, ad - 0.5)
        ll = ll + jnp.sum(sl1 * posf)

    conf = conf_ref[0, :, :]
    rmax = jnp.max(conf, axis=1)
    s = jnp.sum(jnp.exp(conf - rmax[:, None]), axis=1)
    lse = jnp.log(s) + rmax
    cio = lax.broadcasted_iota(jnp.int32, (_P, _NUM_CLASSES), 1)
    conf_t_i = conf_t.astype(jnp.int32)
    g = jnp.sum(jnp.where(cio == conf_t_i[:, None], conf, 0.0), axis=1)
    ce = lse - g

    k = jnp.minimum(_NEGPOS_RATIO * num_pos, jnp.float32(_P - 1))
    k = jnp.minimum(k, jnp.float32(_P) - num_pos)
    ce_m = jnp.where(pos, -2.0, ce)
    lo0 = jnp.float32(-1.0)
    hi0 = jnp.max(ce_m) + 1.0

    def bis(_, carry):
        lo, hi = carry
        mid = 0.5 * (lo + hi)
        c = jnp.sum(jnp.where(ce_m > mid, 1.0, 0.0))
        geq = c >= k
        return jnp.where(geq, mid, lo), jnp.where(geq, hi, mid)

    lo, hi = lax.fori_loop(0, 32, bis, (lo0, hi0))
    cnt_hi = jnp.sum(jnp.where(ce_m > hi, 1.0, 0.0))
    s_top = jnp.sum(jnp.where(ce_m > hi, ce_m, 0.0)) + (k - cnt_hi) * hi
    lc = jnp.sum(ce * posf) + s_top

    @pl.when(b == 0)
    def _init():
        ll_ref[0, 0] = 0.0
        lc_ref[0, 0] = 0.0
        np_ref[0, 0] = 0.0

    ll_ref[0, 0] += ll
    lc_ref[0, 0] += lc
    np_ref[0, 0] += num_pos


@jax.jit
def kernel(loc_data, conf_data, priors, targets):
    loc_t = jnp.transpose(loc_data, (0, 2, 1))
    targets_t = jnp.transpose(targets, (0, 2, 1))
    priors_t = jnp.transpose(priors, (1, 0))

    out_shapes = [jax.ShapeDtypeStruct((1, 1), jnp.float32)] * 3
    scalar_spec = pl.BlockSpec((1, 1), lambda b: (0, 0),
                               memory_space=pltpu.SMEM)
    ll, lc, npos = pl.pallas_call(
        _body,
        grid=(_B,),
        in_specs=[
            pl.BlockSpec((1, 4, _P), lambda b: (b, 0, 0)),
            pl.BlockSpec((1, _P, _NUM_CLASSES), lambda b: (b, 0, 0)),
            pl.BlockSpec((4, _P), lambda b: (0, 0)),
            pl.BlockSpec((1, 5, _O), lambda b: (b, 0, 0)),
        ],
        out_specs=[scalar_spec, scalar_spec, scalar_spec],
        out_shape=out_shapes,
        interpret=_INTERPRET,
    )(loc_t, conf_data, priors_t, targets_t)

    n = npos[0, 0]
    return (ll[0, 0] / n, lc[0, 0] / n)


# speedup vs baseline: 30.6123x; 4.0284x over previous
import jax
import jax.numpy as jnp
from jax import lax
from jax.experimental import pallas as pl
from jax.experimental.pallas import tpu as pltpu

_NUM_CLASSES = 21
_THRESHOLD = 0.5
_NEGPOS_RATIO = 3
_V0 = 0.1
_V1 = 0.2
_B = 32
_P = 8732
_O = 12

_INTERPRET = False


def _body(loc_ref, conf_ref, priors_ref, targets_ref,
          ll_ref, lc_ref, np_ref, ce_ref):
    b = pl.program_id(0)

    px = priors_ref[0, :]
    py = priors_ref[1, :]
    pw = priors_ref[2, :]
    ph = priors_ref[3, :]
    x1p = (px - pw * 0.5)[None, :]
    y1p = (py - ph * 0.5)[None, :]
    x2p = (px + pw * 0.5)[None, :]
    y2p = (py + ph * 0.5)[None, :]
    area_p = (pw * ph)[None, :]

    t = targets_ref[0]
    tx1 = t[:, 0:1]
    ty1 = t[:, 1:2]
    tx2 = t[:, 2:3]
    ty2 = t[:, 3:4]
    area_t = (tx2 - tx1) * (ty2 - ty1)
    iw = jnp.maximum(jnp.minimum(x2p, tx2) - jnp.maximum(x1p, tx1), 0.0)
    ih = jnp.maximum(jnp.minimum(y2p, ty2) - jnp.maximum(y1p, ty1), 0.0)
    inter = iw * ih
    iou = inter / (area_p + area_t - inter)

    oio = lax.broadcasted_iota(jnp.int32, (_O, _P), 0)
    lio = lax.broadcasted_iota(jnp.int32, (_O, _P), 1)

    bv0 = jnp.max(iou, axis=0)
    bti0 = jnp.min(jnp.where(iou == bv0[None, :], oio, _O), axis=0)
    m = jnp.max(iou, axis=1, keepdims=True)
    bpi = jnp.min(jnp.where(iou == m, lio, _P), axis=1, keepdims=True)

    fm = lio == bpi
    f_any = jnp.max(jnp.where(fm, 1, 0), axis=0) > 0
    f_o = jnp.max(jnp.where(fm, oio, -1), axis=0)
    bv = jnp.where(f_any, 2.0, bv0)
    bti = jnp.where(f_any, f_o, bti0)

    onehot = oio == bti[None, :]
    mx1 = jnp.sum(jnp.where(onehot, tx1, 0.0), axis=0)
    my1 = jnp.sum(jnp.where(onehot, ty1, 0.0), axis=0)
    mx2 = jnp.sum(jnp.where(onehot, tx2, 0.0), axis=0)
    my2 = jnp.sum(jnp.where(onehot, ty2, 0.0), axis=0)
    labf = jnp.sum(jnp.where(onehot, t[:, 4:5], 0.0), axis=0)

    conf_t = jnp.where(bv < _THRESHOLD, 0.0, labf + 1.0)
    pos = conf_t > 0.0
    posf = pos.astype(jnp.float32)
    num_pos = jnp.sum(posf)

    gcx = ((mx1 + mx2) * 0.5 - px) / (_V0 * pw)
    gcy = ((my1 + my2) * 0.5 - py) / (_V0 * ph)
    gw = jnp.log((mx2 - mx1) / pw) * (1.0 / _V1)
    gh = jnp.log((my2 - my1) / ph) * (1.0 / _V1)
    ll = jnp.float32(0.0)
    for c, g in enumerate((gcx, gcy, gw, gh)):
        d = loc_ref[0, c, :] - g
        ad = jnp.abs(d)
        sl1 = jnp.where(ad < 1.0, 0.5 * d * d, ad - 0.5)
        ll = ll + jnp.sum(sl1 * posf)

    conf = conf_ref[0]
    rmax = jnp.max(conf, axis=0)
    s = jnp.sum(jnp.exp(conf - rmax[None, :]), axis=0)
    lse = jnp.log(s) + rmax
    cio = lax.broadcasted_iota(jnp.int32, (_NUM_CLASSES, _P), 0)
    conf_t_i = conf_t.astype(jnp.int32)
    g = jnp.sum(jnp.where(cio == conf_t_i[None, :], conf, 0.0), axis=0)
    ce = lse - g

    ce_m = jnp.where(pos, -2.0, ce)
    ce_ref[pl.ds(b, 1), :] = ce_m[None, :]

    @pl.when(b == 0)
    def _init():
        ll_ref[0, 0] = 0.0
        lc_ref[0, 0] = 0.0
        np_ref[0, 0] = 0.0

    ll_ref[0, 0] += ll
    lc_ref[0, 0] += jnp.sum(ce * posf)
    np_ref[0, 0] += num_pos

    @pl.when(b == _B - 1)
    def _mine():
        ce_all = ce_ref[...]
        npos = jnp.sum(jnp.where(ce_all == -2.0, 1.0, 0.0),
                       axis=1, keepdims=True)
        k = jnp.minimum(_NEGPOS_RATIO * npos, jnp.float32(_P - 1))
        k = jnp.minimum(k, jnp.float32(_P) - npos)
        lo0 = jnp.full((_B, 1), -1.0, jnp.float32)
        hi0 = jnp.max(ce_all, axis=1, keepdims=True) + 1.0

        def bis(_, carry):
            lo, hi = carry
            mid = 0.5 * (lo + hi)
            c = jnp.sum(jnp.where(ce_all > mid, 1.0, 0.0),
                        axis=1, keepdims=True)
            geq = c >= k
            return jnp.where(geq, mid, lo), jnp.where(geq, hi, mid)

        lo, hi = lax.fori_loop(0, 32, bis, (lo0, hi0))
        cnt_hi = jnp.sum(jnp.where(ce_all > hi, 1.0, 0.0),
                         axis=1, keepdims=True)
        s_top = (jnp.sum(jnp.where(ce_all > hi, ce_all, 0.0),
                         axis=1, keepdims=True)
                 + (k - cnt_hi) * hi)
        lc_ref[0, 0] += jnp.sum(s_top)


@jax.jit
def kernel(loc_data, conf_data, priors, targets):
    loc_t = jnp.transpose(loc_data, (0, 2, 1))
    conf_t2 = jnp.transpose(conf_data, (0, 2, 1))
    priors_t = jnp.transpose(priors, (1, 0))

    out_shapes = [jax.ShapeDtypeStruct((1, 1), jnp.float32)] * 3
    scalar_spec = pl.BlockSpec((1, 1), lambda b: (0, 0),
                               memory_space=pltpu.SMEM)
    ll, lc, npos = pl.pallas_call(
        _body,
        grid=(_B,),
        in_specs=[
            pl.BlockSpec((1, 4, _P), lambda b: (b, 0, 0)),
            pl.BlockSpec((1, _NUM_CLASSES, _P), lambda b: (b, 0, 0)),
            pl.BlockSpec((4, _P), lambda b: (0, 0)),
            pl.BlockSpec((1, _O, 5), lambda b: (b, 0, 0)),
        ],
        out_specs=[scalar_spec, scalar_spec, scalar_spec],
        out_shape=out_shapes,
        scratch_shapes=[pltpu.VMEM((_B, _P), jnp.float32)],
        interpret=_INTERPRET,
    )(loc_t, conf_t2, priors_t, targets)

    n = npos[0, 0]
    return (ll[0, 0] / n, lc[0, 0] / n)


# speedup vs baseline: 38.5468x; 1.2592x over previous
import jax
import jax.numpy as jnp
from jax import lax
from jax.experimental import pallas as pl
from jax.experimental.pallas import tpu as pltpu

_NUM_CLASSES = 21
_THRESHOLD = 0.5
_NEGPOS_RATIO = 3
_V0 = 0.1
_V1 = 0.2
_B = 32
_P = 8732
_O = 12

_INTERPRET = False


def _body(loc_ref, conf_ref, paug_ref, targets_ref, targets_t_ref,
          ll_ref, lc_ref, np_ref, ce_ref):
    b = pl.program_id(0)

    x1p = paug_ref[0:1, :]
    y1p = paug_ref[1:2, :]
    x2p = paug_ref[2:3, :]
    y2p = paug_ref[3:4, :]
    area_p = paug_ref[4:5, :]
    px = paug_ref[5, :]
    py = paug_ref[6, :]
    inv_v0pw = paug_ref[7, :]
    inv_v0ph = paug_ref[8, :]
    inv_pw = paug_ref[9, :]
    inv_ph = paug_ref[10, :]

    t = targets_ref[0]
    tx1 = t[:, 0:1]
    ty1 = t[:, 1:2]
    tx2 = t[:, 2:3]
    ty2 = t[:, 3:4]
    area_t = (tx2 - tx1) * (ty2 - ty1)
    iw = jnp.maximum(jnp.minimum(x2p, tx2) - jnp.maximum(x1p, tx1), 0.0)
    ih = jnp.maximum(jnp.minimum(y2p, ty2) - jnp.maximum(y1p, ty1), 0.0)
    inter = iw * ih
    iou = inter / (area_p + area_t - inter)

    oio = lax.broadcasted_iota(jnp.int32, (_O, _P), 0)
    lio = lax.broadcasted_iota(jnp.int32, (_O, _P), 1)

    bv0 = jnp.max(iou, axis=0)
    bti0 = jnp.min(jnp.where(iou == bv0[None, :], oio, _O), axis=0)
    m = jnp.max(iou, axis=1, keepdims=True)
    bpi = jnp.min(jnp.where(iou == m, lio, _P), axis=1, keepdims=True)

    fm = lio == bpi
    f_o = jnp.max(jnp.where(fm, oio, -1), axis=0)
    f_any = f_o >= 0
    bv = jnp.where(f_any, 2.0, bv0)
    bti = jnp.where(f_any, f_o, bti0)

    onehotf = (oio == bti[None, :]).astype(jnp.float32)
    matched = lax.dot_general(
        targets_t_ref[0], onehotf, (((1,), (0,)), ((), ())),
        preferred_element_type=jnp.float32)
    mx1 = matched[0, :]
    my1 = matched[1, :]
    mx2 = matched[2, :]
    my2 = matched[3, :]
    labf = matched[4, :]

    conf_t = jnp.where(bv < _THRESHOLD, 0.0, labf + 1.0)
    pos = conf_t > 0.0
    posf = pos.astype(jnp.float32)

    gcx = ((mx1 + mx2) * 0.5 - px) * inv_v0pw
    gcy = ((my1 + my2) * 0.5 - py) * inv_v0ph
    gw = jnp.log((mx2 - mx1) * inv_pw) * (1.0 / _V1)
    gh = jnp.log((my2 - my1) * inv_ph) * (1.0 / _V1)
    ll = jnp.float32(0.0)
    for c, g in enumerate((gcx, gcy, gw, gh)):
        d = loc_ref[0, c, :] - g
        ad = jnp.abs(d)
        sl1 = jnp.where(ad < 1.0, 0.5 * d * d, ad - 0.5)
        ll = ll + jnp.sum(sl1 * posf)

    conf = conf_ref[0]
    rmax = jnp.max(conf, axis=0)
    s = jnp.sum(jnp.exp(conf - rmax[None, :]), axis=0)
    lse = jnp.log(s) + rmax
    cio = lax.broadcasted_iota(jnp.int32, (_NUM_CLASSES, _P), 0)
    conf_t_i = conf_t.astype(jnp.int32)
    g = jnp.sum(jnp.where(cio == conf_t_i[None, :], conf, 0.0), axis=0)
    ce = lse - g

    ce_m = jnp.where(pos, -2.0, ce)
    ce_ref[pl.ds(b, 1), :] = ce_m[None, :]

    @pl.when(b == 0)
    def _init():
        ll_ref[0, 0] = 0.0
        lc_ref[0, 0] = 0.0

    ll_ref[0, 0] += ll
    lc_ref[0, 0] += jnp.sum(ce * posf)

    @pl.when(b == _B - 1)
    def _mine():
        ce_all = ce_ref[...]
        npos = jnp.sum(jnp.where(ce_all == -2.0, 1.0, 0.0),
                       axis=1, keepdims=True)
        k = jnp.minimum(_NEGPOS_RATIO * npos, jnp.float32(_P - 1))
        k = jnp.minimum(k, jnp.float32(_P) - npos)
        lo0 = jnp.full((_B, 1), -1.0, jnp.float32)
        hi0 = jnp.max(ce_all, axis=1, keepdims=True) + 1.0

        def bis(_, carry):
            lo, hi = carry
            mid = 0.5 * (lo + hi)
            c = jnp.sum(jnp.where(ce_all > mid, 1.0, 0.0),
                        axis=1, keepdims=True)
            geq = c >= k
            return jnp.where(geq, mid, lo), jnp.where(geq, hi, mid)

        lo, hi = lax.fori_loop(0, 24, bis, (lo0, hi0))
        cnt_hi = jnp.sum(jnp.where(ce_all > hi, 1.0, 0.0),
                         axis=1, keepdims=True)
        s_top = (jnp.sum(jnp.where(ce_all > hi, ce_all, 0.0),
                         axis=1, keepdims=True)
                 + (k - cnt_hi) * hi)
        lc_ref[0, 0] += jnp.sum(s_top)
        np_ref[0, 0] = jnp.sum(npos)


@jax.jit
def kernel(loc_data, conf_data, priors, targets):
    loc_t = jnp.transpose(loc_data, (0, 2, 1))
    conf_t2 = jnp.transpose(conf_data, (0, 2, 1))
    targets_t = jnp.transpose(targets, (0, 2, 1))

    px, py, pw, ph = priors[:, 0], priors[:, 1], priors[:, 2], priors[:, 3]
    paug = jnp.stack([
        px - pw * 0.5, py - ph * 0.5, px + pw * 0.5, py + ph * 0.5,
        pw * ph, px, py,
        1.0 / (_V0 * pw), 1.0 / (_V0 * ph), 1.0 / pw, 1.0 / ph,
    ], axis=0)

    out_shapes = [jax.ShapeDtypeStruct((1, 1), jnp.float32)] * 3
    scalar_spec = pl.BlockSpec((1, 1), lambda b: (0, 0),
                               memory_space=pltpu.SMEM)
    ll, lc, npos = pl.pallas_call(
        _body,
        grid=(_B,),
        in_specs=[
            pl.BlockSpec((1, 4, _P), lambda b: (b, 0, 0)),
            pl.BlockSpec((1, _NUM_CLASSES, _P), lambda b: (b, 0, 0)),
            pl.BlockSpec((11, _P), lambda b: (0, 0)),
            pl.BlockSpec((1, _O, 5), lambda b: (b, 0, 0)),
            pl.BlockSpec((1, 5, _O), lambda b: (b, 0, 0)),
        ],
        out_specs=[scalar_spec, scalar_spec, scalar_spec],
        out_shape=out_shapes,
        scratch_shapes=[pltpu.VMEM((_B, _P), jnp.float32)],
        interpret=_INTERPRET,
    )(loc_t, conf_t2, paug, targets, targets_t)

    n = npos[0, 0]
    return (ll[0, 0] / n, lc[0, 0] / n)
